# Initial kernel scaffold; baseline (speedup 1.0000x reference)
import jax
import jax.numpy as jnp
from jax.experimental import pallas as pl


def kernel(x, position_ids, cos_cached, sin_cached):
    raise NotImplementedError("write your pallas kernel here")


# speedup vs baseline: 1.5733x; 1.5733x over previous
import functools

import jax
import jax.numpy as jnp
from jax import lax
from jax.experimental import pallas as pl
from jax.experimental.pallas import tpu as pltpu
from jax.experimental.pallas import tpu_sc as plsc

DIM = 128
_NC, _NS = 2, 16
_NW = _NC * _NS
_CHUNK = 128


@functools.lru_cache(maxsize=None)
def _build_sc_gather(n_rows: int):
    assert n_rows % (8 * _NW) == 0
    b_per_w = n_rows // _NW
    n_chunks = b_per_w // _CHUNK
    mesh = plsc.VectorSubcoreMesh(core_axis_name="c", subcore_axis_name="s")

    @functools.partial(
        pl.kernel,
        mesh=mesh,
        out_type=[
            jax.ShapeDtypeStruct((n_rows, DIM), jnp.float32),
            jax.ShapeDtypeStruct((n_rows, DIM), jnp.float32),
        ],
        scratch_types=[
            pltpu.VMEM((b_per_w,), jnp.int32),
            pltpu.VMEM((b_per_w, DIM), jnp.float32),
            pltpu.SemaphoreType.DMA,
        ],
    )
    def sc_gather(pos_hbm, cos_hbm, sin_hbm, cos_out, sin_out,
                  idx_v, rows_v, sem):
        wid = lax.axis_index("s") * _NC + lax.axis_index("c")
        base = wid * b_per_w
        pltpu.sync_copy(pos_hbm.at[pl.ds(base, b_per_w)], idx_v)
        for table, out in ((cos_hbm, cos_out), (sin_hbm, sin_out)):
            cps = []
            for j in range(n_chunks):
                sl = pl.ds(j * _CHUNK, _CHUNK)
                cps.append(pltpu.async_copy(table.at[idx_v.at[sl]],
                                            rows_v.at[sl], sem))
            for cp in cps:
                cp.wait()
            pltpu.sync_copy(rows_v, out.at[pl.ds(base, b_per_w)])

    return sc_gather


def kernel(x, position_ids, cos_cached, sin_cached):
    b, s = position_ids.shape
    pos = position_ids.reshape(-1).astype(jnp.int32)
    cos_flat, sin_flat = _build_sc_gather(b * s)(
        pos, cos_cached.astype(jnp.float32), sin_cached.astype(jnp.float32))
    return (cos_flat.reshape(b, 1, s, DIM).astype(x.dtype),
            sin_flat.reshape(b, 1, s, DIM).astype(x.dtype))
